# Initial kernel scaffold; baseline (speedup 1.0000x reference)
import jax
import jax.numpy as jnp
from jax.experimental import pallas as pl


def kernel(h, mem, W_q, b_q):
    raise NotImplementedError("write your pallas kernel here")


# speedup vs baseline: 2.1975x; 2.1975x over previous
import functools
import math

import jax
import jax.numpy as jnp
from jax import lax
from jax.experimental import pallas as pl
from jax.experimental.pallas import tpu as pltpu
from jax.experimental.pallas import tpu_sc as plsc

_K = 10
_QB = 256
_MT = 2048
_RUN = 128
_NEG = float("-inf")
_IMAX = 2147483647


def _topk_body(h_ref, w_ref, b_ref, mem_ref, msq_ref, idx_ref, wgt_ref,
               q_ref, tv_ref, ti_ref, *, nm, m_total, mt):
    mb = pl.program_id(1)

    @pl.when(mb == 0)
    def _():
        q = jnp.dot(h_ref[...].astype(jnp.bfloat16),
                    w_ref[...].astype(jnp.bfloat16),
                    preferred_element_type=jnp.float32)
        q_ref[...] = q + b_ref[...][None, :]
        tv_ref[...] = jnp.full(tv_ref.shape, _NEG, jnp.float32)
        ti_ref[...] = jnp.full(ti_ref.shape, _IMAX, jnp.int32)

    q = q_ref[...]
    q_sq = jnp.sum(q * q, axis=1, keepdims=True)
    mem_t = mem_ref[...]
    s = lax.dot_general(q.astype(jnp.bfloat16), mem_t.astype(jnp.bfloat16),
                        (((1,), (1,)), ((), ())),
                        preferred_element_type=jnp.float32)
    msq = msq_ref[...]
    neg = -((q_sq - 2.0 * s) + msq)
    gcol = mb * mt + lax.broadcasted_iota(jnp.int32, neg.shape, 1)
    neg = jnp.where(gcol < m_total, neg, _NEG)

    ext_v = jnp.concatenate([tv_ref[...], neg], axis=1)
    ext_i = jnp.concatenate([ti_ref[...], gcol], axis=1)
    vals, idxs = [], []
    for _ in range(_K):
        mx = jnp.max(ext_v, axis=1, keepdims=True)
        sel = jnp.min(jnp.where(ext_v == mx, ext_i, _IMAX), axis=1,
                      keepdims=True)
        vals.append(mx)
        idxs.append(sel)
        ext_v = jnp.where(ext_i == sel, _NEG, ext_v)
    new_tv = jnp.concatenate(vals, axis=1)
    new_ti = jnp.concatenate(idxs, axis=1)
    tv_ref[:, :_K] = new_tv
    ti_ref[:, :_K] = new_ti

    @pl.when(mb == nm - 1)
    def _():
        e = jnp.exp(new_tv - new_tv[:, 0:1])
        wgt_ref[...] = e / jnp.sum(e, axis=1, keepdims=True)
        idx_ref[...] = new_ti


def _topk_pallas(h, mem, W_q, b_q, interpret=False):
    Q, DH = h.shape
    M, D = mem.shape
    nq = Q // _QB
    nm = math.ceil(M / _MT)
    mpad = nm * _MT
    msq = jnp.pad(jnp.sum(mem * mem, axis=1), (0, mpad - M))[None, :]
    body = functools.partial(_topk_body, nm=nm, m_total=M, mt=_MT)
    return pl.pallas_call(
        body,
        grid=(nq, nm),
        in_specs=[
            pl.BlockSpec((_QB, DH), lambda i, j: (i, 0)),
            pl.BlockSpec((DH, D), lambda i, j: (0, 0)),
            pl.BlockSpec((D,), lambda i, j: (0,)),
            pl.BlockSpec((_MT, D), lambda i, j: (j, 0)),
            pl.BlockSpec((1, _MT), lambda i, j: (0, j)),
        ],
        out_specs=[
            pl.BlockSpec((_QB, _K), lambda i, j: (i, 0)),
            pl.BlockSpec((_QB, _K), lambda i, j: (i, 0)),
        ],
        out_shape=[
            jax.ShapeDtypeStruct((Q, _K), jnp.int32),
            jax.ShapeDtypeStruct((Q, _K), jnp.float32),
        ],
        scratch_shapes=[
            pltpu.VMEM((_QB, D), jnp.float32),
            pltpu.VMEM((_QB, _RUN), jnp.float32),
            pltpu.VMEM((_QB, _RUN), jnp.int32),
        ],
        compiler_params=pltpu.CompilerParams(
            dimension_semantics=("arbitrary", "arbitrary"),
        ),
        interpret=interpret,
    )(h, W_q, b_q, mem, msq)


def _gather_sc(mem, idx, wgt):
    M, D = mem.shape
    Q, K = idx.shape
    info = plsc.get_sparse_core_info()
    nw = info.num_cores * info.num_subcores
    qw = Q // nw
    rows_w = qw * K
    ch = 4
    rch = rows_w // ch
    nd = D // 16

    idx1 = idx.reshape(Q * K)
    wgt16 = jnp.pad(wgt, ((0, 0), (0, 16 - K))).reshape(Q * 16)
    wpw = qw * 16
    mesh = plsc.VectorSubcoreMesh(core_axis_name="c", subcore_axis_name="s")

    @functools.partial(
        pl.kernel,
        out_type=jax.ShapeDtypeStruct((Q, D), jnp.float32),
        mesh=mesh,
        scratch_types=[
            pltpu.VMEM((ch, rch), jnp.int32),
            pltpu.VMEM((wpw,), jnp.float32),
            pltpu.VMEM((ch, rch, D), jnp.float32),
            pltpu.VMEM((qw, D), jnp.float32),
            pltpu.SemaphoreType.DMA,
        ],
    )
    def gather_kernel(mem_hbm, idx_hbm, wgt_hbm, out_hbm,
                      idx_v, wgt_v, rows_v, out_v, sem):
        wid = lax.axis_index("s") * info.num_cores + lax.axis_index("c")
        for c in range(ch):
            pltpu.sync_copy(idx_hbm.at[pl.ds(wid * rows_w + c * rch, rch)],
                            idx_v.at[c])
        pltpu.sync_copy(wgt_hbm.at[pl.ds(wid * wpw, wpw)], wgt_v)
        copies = [
            pltpu.async_copy(mem_hbm.at[idx_v.at[c]], rows_v.at[c], sem)
            for c in range(ch)
        ]
        for c in range(ch):
            copies[c].wait()

        def qbody(qq, carry):
            base = qq * K
            w16 = wgt_v[pl.ds(qq * 16, 16)]
            accs = [jnp.zeros((16,), jnp.float32) for _ in range(nd)]
            for k in range(K):
                i = base + k
                c = i // rch
                r = i - c * rch
                w = w16[k]
                for d in range(nd):
                    accs[d] = accs[d] + w * rows_v[c, r, pl.ds(d * 16, 16)]
            for d in range(nd):
                out_v[qq, pl.ds(d * 16, 16)] = accs[d]
            return carry

        lax.fori_loop(0, qw, qbody, 0)
        pltpu.sync_copy(out_v, out_hbm.at[pl.ds(wid * qw, qw)])

    return gather_kernel(mem, idx1, wgt16)


def kernel(h, mem, W_q, b_q):
    idx, wgt = _topk_pallas(h, mem, W_q, b_q)
    retrieved = _gather_sc(mem, idx, wgt)
    return retrieved, idx


# speedup vs baseline: 2.6339x; 1.1986x over previous
import functools
import math

import jax
import jax.numpy as jnp
from jax import lax
from jax.experimental import pallas as pl
from jax.experimental.pallas import tpu as pltpu
from jax.experimental.pallas import tpu_sc as plsc

_K = 10
_QB = 256
_MT = 2048
_RUN = 128
_NEG = float("-inf")
_IMAX = 2147483647


def _topk_body(h_ref, w_ref, b_ref, mem_ref, msq_ref, idx_ref, wgt_ref,
               q_ref, tv_ref, ti_ref, *, nm, m_total, mt):
    mb = pl.program_id(1)

    @pl.when(mb == 0)
    def _():
        q = jnp.dot(h_ref[...].astype(jnp.bfloat16),
                    w_ref[...].astype(jnp.bfloat16),
                    preferred_element_type=jnp.float32)
        q_ref[...] = q + b_ref[...][None, :]
        tv_ref[...] = jnp.full(tv_ref.shape, _NEG, jnp.float32)
        ti_ref[...] = jnp.full(ti_ref.shape, _IMAX, jnp.int32)

    q = q_ref[...]
    q_sq = jnp.sum(q * q, axis=1, keepdims=True)
    mem_t = mem_ref[...]
    s = lax.dot_general(q.astype(jnp.bfloat16), mem_t.astype(jnp.bfloat16),
                        (((1,), (1,)), ((), ())),
                        preferred_element_type=jnp.float32)
    msq = msq_ref[...]
    neg = -((q_sq - 2.0 * s) + msq)
    iota = lax.broadcasted_iota(jnp.int32, neg.shape, 1)
    neg = jnp.where(mb * mt + iota < m_total, neg, _NEG)

    lane16 = lax.broadcasted_iota(jnp.int32, tv_ref.shape, 1)

    def _cond(carry):
        _, mx, rv, _ = carry
        return jnp.any(mx > rv[:, _K - 1:_K])

    def _body(carry):
        cand, mx, rv, ri = carry
        live = mx > rv[:, _K - 1:_K]
        sel = jnp.min(jnp.where(cand == mx, iota, mt), axis=1,
                      keepdims=True)
        gsel = mb * mt + sel
        pos = jnp.sum((rv >= mx).astype(jnp.int32), axis=1,
                      keepdims=True)
        sh_v = jnp.concatenate([rv[:, :1], rv[:, :-1]], axis=1)
        sh_i = jnp.concatenate([ri[:, :1], ri[:, :-1]], axis=1)
        nv = jnp.where(lane16 < pos, rv, jnp.where(lane16 == pos, mx, sh_v))
        ni = jnp.where(lane16 < pos, ri, jnp.where(lane16 == pos, gsel, sh_i))
        rv = jnp.where(live, nv, rv)
        ri = jnp.where(live, ni, ri)
        cand = jnp.where(iota == sel, _NEG, cand)
        mx = jnp.max(cand, axis=1, keepdims=True)
        return cand, mx, rv, ri

    mx0 = jnp.max(neg, axis=1, keepdims=True)
    _, _, run_v, run_i = lax.while_loop(
        _cond, _body, (neg, mx0, tv_ref[...], ti_ref[...]))
    tv_ref[...] = run_v
    ti_ref[...] = run_i

    @pl.when(mb == nm - 1)
    def _():
        tv = run_v[:, :_K]
        e = jnp.exp(tv - tv[:, 0:1])
        wgt_ref[...] = e / jnp.sum(e, axis=1, keepdims=True)
        idx_ref[...] = run_i[:, :_K]


def _topk_pallas(h, mem, W_q, b_q, interpret=False):
    Q, DH = h.shape
    M, D = mem.shape
    nq = Q // _QB
    nm = math.ceil(M / _MT)
    mpad = nm * _MT
    msq = jnp.pad(jnp.sum(mem * mem, axis=1), (0, mpad - M))[None, :]
    body = functools.partial(_topk_body, nm=nm, m_total=M, mt=_MT)
    return pl.pallas_call(
        body,
        grid=(nq, nm),
        in_specs=[
            pl.BlockSpec((_QB, DH), lambda i, j: (i, 0)),
            pl.BlockSpec((DH, D), lambda i, j: (0, 0)),
            pl.BlockSpec((D,), lambda i, j: (0,)),
            pl.BlockSpec((_MT, D), lambda i, j: (j, 0)),
            pl.BlockSpec((1, _MT), lambda i, j: (0, j)),
        ],
        out_specs=[
            pl.BlockSpec((_QB, _K), lambda i, j: (i, 0)),
            pl.BlockSpec((_QB, _K), lambda i, j: (i, 0)),
        ],
        out_shape=[
            jax.ShapeDtypeStruct((Q, _K), jnp.int32),
            jax.ShapeDtypeStruct((Q, _K), jnp.float32),
        ],
        scratch_shapes=[
            pltpu.VMEM((_QB, D), jnp.float32),
            pltpu.VMEM((_QB, 16), jnp.float32),
            pltpu.VMEM((_QB, 16), jnp.int32),
        ],
        compiler_params=pltpu.CompilerParams(
            dimension_semantics=("arbitrary", "arbitrary"),
        ),
        interpret=interpret,
    )(h, W_q, b_q, mem, msq)


def _gather_sc(mem, idx, wgt):
    M, D = mem.shape
    Q, K = idx.shape
    info = plsc.get_sparse_core_info()
    nw = info.num_cores * info.num_subcores
    qw = Q // nw
    rows_w = qw * K
    ch = 4
    rch = rows_w // ch
    nd = D // 16

    idx1 = idx.reshape(Q * K)
    wgt16 = jnp.pad(wgt, ((0, 0), (0, 16 - K))).reshape(Q * 16)
    wpw = qw * 16
    mesh = plsc.VectorSubcoreMesh(core_axis_name="c", subcore_axis_name="s")

    @functools.partial(
        pl.kernel,
        out_type=jax.ShapeDtypeStruct((Q, D), jnp.float32),
        mesh=mesh,
        scratch_types=[
            pltpu.VMEM((ch, rch), jnp.int32),
            pltpu.VMEM((wpw,), jnp.float32),
            pltpu.VMEM((ch, rch, D), jnp.float32),
            pltpu.VMEM((qw, D), jnp.float32),
            pltpu.SemaphoreType.DMA,
        ],
    )
    def gather_kernel(mem_hbm, idx_hbm, wgt_hbm, out_hbm,
                      idx_v, wgt_v, rows_v, out_v, sem):
        wid = lax.axis_index("s") * info.num_cores + lax.axis_index("c")
        for c in range(ch):
            pltpu.sync_copy(idx_hbm.at[pl.ds(wid * rows_w + c * rch, rch)],
                            idx_v.at[c])
        pltpu.sync_copy(wgt_hbm.at[pl.ds(wid * wpw, wpw)], wgt_v)
        copies = [
            pltpu.async_copy(mem_hbm.at[idx_v.at[c]], rows_v.at[c], sem)
            for c in range(ch)
        ]
        for c in range(ch):
            copies[c].wait()

        def qbody(qq, carry):
            base = qq * K
            w16 = wgt_v[pl.ds(qq * 16, 16)]
            accs = [jnp.zeros((16,), jnp.float32) for _ in range(nd)]
            for k in range(K):
                i = base + k
                c = i // rch
                r = i - c * rch
                w = w16[k]
                for d in range(nd):
                    accs[d] = accs[d] + w * rows_v[c, r, pl.ds(d * 16, 16)]
            for d in range(nd):
                out_v[qq, pl.ds(d * 16, 16)] = accs[d]
            return carry

        lax.fori_loop(0, qw, qbody, 0)
        pltpu.sync_copy(out_v, out_hbm.at[pl.ds(wid * qw, qw)])

    return gather_kernel(mem, idx1, wgt16)


def kernel(h, mem, W_q, b_q):
    idx, wgt = _topk_pallas(h, mem, W_q, b_q)
    retrieved = _gather_sc(mem, idx, wgt)
    return retrieved, idx
